# Initial kernel scaffold; baseline (speedup 1.0000x reference)
import jax
import jax.numpy as jnp
from jax.experimental import pallas as pl


def kernel(atomic_numbers, pair_indices, f_ij, emb_table, W_rbf, b_rbf, W_out, b_out):
    raise NotImplementedError("write your pallas kernel here")


# speedup vs baseline: 5.7660x; 5.7660x over previous
import functools

import jax
import jax.numpy as jnp
from jax import lax
from jax.experimental import pallas as pl
from jax.experimental.pallas import tpu as pltpu
from jax.experimental.pallas import tpu_sc as plsc

N_NODES = 10000
N_EDGES = 320000
EMB = 128
NUM_EMB = 95
LANES = 16
NC, NS = 2, 16
NW = NC * NS
PER_W = N_EDGES // NW
CB = 80
NCHUNK = PER_W // CB

NODE_BLK = 2000
EDGE_BLK = 4000


def _silu(x):
    return x / (1.0 + jnp.exp(-x))


def _node_tables_body(an_ref, emb_ref, w1_ref, w2_ref, g_ref, h_ref):
    z = an_ref[0, 0, :]
    col = lax.broadcasted_iota(jnp.int32, (NODE_BLK, EMB), 1)
    onehot = (z[:, None] == col).astype(jnp.float32)
    x = jnp.dot(onehot, emb_ref[...], preferred_element_type=jnp.float32)
    g_ref[...] = jnp.dot(x, w1_ref[...], preferred_element_type=jnp.float32)
    h_ref[...] = jnp.dot(x, w2_ref[...], preferred_element_type=jnp.float32)


def _node_tables(an3, emb_pad, w1, w2):
    n_blk = N_NODES // NODE_BLK
    return pl.pallas_call(
        _node_tables_body,
        grid=(n_blk,),
        in_specs=[
            pl.BlockSpec((1, 1, NODE_BLK), lambda i: (i, 0, 0)),
            pl.BlockSpec((EMB, EMB), lambda i: (0, 0)),
            pl.BlockSpec((EMB, EMB), lambda i: (0, 0)),
            pl.BlockSpec((EMB, EMB), lambda i: (0, 0)),
        ],
        out_specs=[
            pl.BlockSpec((NODE_BLK, EMB), lambda i: (i, 0)),
            pl.BlockSpec((NODE_BLK, EMB), lambda i: (i, 0)),
        ],
        out_shape=[
            jax.ShapeDtypeStruct((N_NODES, EMB), jnp.float32),
            jax.ShapeDtypeStruct((N_NODES, EMB), jnp.float32),
        ],
    )(an3, emb_pad, w1, w2)


def _sc_gather_sum_body(g_hbm, h_hbm, pi_hbm, pj_hbm, s_hbm,
                        idxi, idxj, ri, rj, semi, semj):
    wid = lax.axis_index("s") * NC + lax.axis_index("c")
    base0 = wid * PER_W

    def chunk(t, carry):
        base = base0 + t * CB
        pltpu.sync_copy(pi_hbm.at[pl.ds(base, CB)], idxi)
        pltpu.sync_copy(pj_hbm.at[pl.ds(base, CB)], idxj)
        cpi = pltpu.async_copy(g_hbm.at[idxi], ri, semi)
        cpj = pltpu.async_copy(h_hbm.at[idxj], rj, semj)
        cpi.wait()
        cpj.wait()

        def row(e, c2):
            for c in range(EMB // LANES):
                sl = pl.ds(c * LANES, LANES)
                ri[e, sl] = ri[e, sl] + rj[e, sl]
            return c2

        lax.fori_loop(0, CB, row, 0, unroll=False)
        pltpu.sync_copy(ri, s_hbm.at[pl.ds(base, CB)])
        return carry

    lax.fori_loop(0, NCHUNK, chunk, 0, unroll=False)


def _sc_gather_sum(g, h, pi, pj):
    mesh = plsc.VectorSubcoreMesh(
        core_axis_name="c", subcore_axis_name="s", num_cores=NC, num_subcores=NS
    )
    return pl.kernel(
        _sc_gather_sum_body,
        out_type=jax.ShapeDtypeStruct((N_EDGES, EMB), jnp.float32),
        mesh=mesh,
        scratch_types=[
            pltpu.VMEM((CB,), jnp.int32),
            pltpu.VMEM((CB,), jnp.int32),
            pltpu.VMEM((CB, EMB), jnp.float32),
            pltpu.VMEM((CB, EMB), jnp.float32),
            pltpu.SemaphoreType.DMA,
            pltpu.SemaphoreType.DMA,
        ],
    )(g, h, pi, pj)


def _edge_mlp_body(f_ref, s_ref, wr_ref, br_ref, w3_ref, bo_ref, o_ref):
    t = jnp.dot(f_ref[...], wr_ref[...], preferred_element_type=jnp.float32)
    t = _silu(_silu(t + br_ref[...]))
    r = jnp.dot(t, w3_ref[...], preferred_element_type=jnp.float32)
    o_ref[...] = _silu(r + bo_ref[...] + s_ref[...])


def _edge_mlp(f_pad, s, wr_pad, br, w3, bo):
    n_blk = N_EDGES // EDGE_BLK
    return pl.pallas_call(
        _edge_mlp_body,
        grid=(n_blk,),
        in_specs=[
            pl.BlockSpec((EDGE_BLK, 8), lambda i: (i, 0)),
            pl.BlockSpec((EDGE_BLK, EMB), lambda i: (i, 0)),
            pl.BlockSpec((8, EMB), lambda i: (0, 0)),
            pl.BlockSpec((1, EMB), lambda i: (0, 0)),
            pl.BlockSpec((EMB, EMB), lambda i: (0, 0)),
            pl.BlockSpec((1, EMB), lambda i: (0, 0)),
        ],
        out_specs=pl.BlockSpec((EDGE_BLK, EMB), lambda i: (i, 0)),
        out_shape=jax.ShapeDtypeStruct((N_EDGES, EMB), jnp.float32),
    )(f_pad, s, wr_pad, br, w3, bo)


def kernel(atomic_numbers, pair_indices, f_ij, emb_table, W_rbf, b_rbf, W_out, b_out):
    an3 = atomic_numbers.astype(jnp.int32).reshape(
        N_NODES // NODE_BLK, 1, NODE_BLK)
    emb_pad = jnp.zeros((EMB, EMB), jnp.float32).at[:NUM_EMB, :].set(emb_table)
    w1 = W_out[:EMB]
    w2 = W_out[EMB:2 * EMB]
    w3 = W_out[2 * EMB:]
    f_pad = jnp.zeros((N_EDGES, 8), jnp.float32).at[:, :f_ij.shape[1]].set(f_ij)
    wr_pad = jnp.zeros((8, EMB), jnp.float32).at[:W_rbf.shape[0], :].set(W_rbf)
    br = b_rbf.reshape(1, EMB)
    bo = b_out.reshape(1, EMB)
    pi = pair_indices[0].astype(jnp.int32)
    pj = pair_indices[1].astype(jnp.int32)

    g, h = _node_tables(an3, emb_pad, w1, w2)
    s = _sc_gather_sum(g, h, pi, pj)
    return _edge_mlp(f_pad, s, wr_pad, br, w3, bo)
